# Initial kernel scaffold; baseline (speedup 1.0000x reference)
import jax
import jax.numpy as jnp
from jax.experimental import pallas as pl


def kernel(song_ids, album_ids, artist_ids, num_features, song_table, album_table, artist_table, W_num, b_num, alpha, bias):
    raise NotImplementedError("write your pallas kernel here")


# speedup vs baseline: 1.4664x; 1.4664x over previous
import functools
import math

import jax
import jax.numpy as jnp
from jax import lax
from jax.experimental import pallas as pl
from jax.experimental.pallas import tpu as pltpu
from jax.experimental.pallas import tpu_sc as plsc

_DS, _DA, _DR = 64, 32, 32
_DM = _DS + _DA + _DR
_NF = 26
_B, _L = 4096, 50
_N = _B * _L
_EPS = 1e-6

_NC, _NSUB = 2, 16
_NW = _NC * _NSUB
_IDS_PER_ROW = 128
_NROWS = _N // _IDS_PER_ROW
_RPW = _NROWS // _NW
_CROWS = 5
_NCH = _RPW // _CROWS
_CTOK = _CROWS * _IDS_PER_ROW


def _sc_gather(sid2, aid2, rid2, song_table, album_table, artist_table):
    mesh = plsc.VectorSubcoreMesh(core_axis_name="c", subcore_axis_name="s")

    @functools.partial(
        pl.kernel,
        mesh=mesh,
        compiler_params=pltpu.CompilerParams(use_tc_tiling_on_sc=False),
        out_type=(
            jax.ShapeDtypeStruct((_N, _DS), jnp.float32),
            jax.ShapeDtypeStruct((_N, _DA), jnp.float32),
            jax.ShapeDtypeStruct((_N, _DR), jnp.float32),
        ),
        scratch_types=[
            pltpu.VMEM((_CTOK,), jnp.int32),
            pltpu.VMEM((_CTOK,), jnp.int32),
            pltpu.VMEM((_CTOK,), jnp.int32),
            pltpu.VMEM((_CTOK, _DS), jnp.float32),
            pltpu.VMEM((_CTOK, _DA), jnp.float32),
            pltpu.VMEM((_CTOK, _DR), jnp.float32),
            pltpu.SemaphoreType.DMA,
        ],
    )
    def k(sid_h, aid_h, rid_h, st_h, at_h, rt_h, os_h, oa_h, or_h,
          idx_s, idx_a, idx_r, rows_s, rows_a, rows_r, sem):
        wid = lax.axis_index("s") * _NC + lax.axis_index("c")

        def body(c, carry):
            tbase = (wid * _NCH + c) * _CTOK
            pltpu.sync_copy(sid_h.at[pl.ds(tbase, _CTOK)], idx_s)
            pltpu.sync_copy(aid_h.at[pl.ds(tbase, _CTOK)], idx_a)
            pltpu.sync_copy(rid_h.at[pl.ds(tbase, _CTOK)], idx_r)
            handles = []
            for j in range(_CROWS):
                sl = pl.ds(j * _IDS_PER_ROW, _IDS_PER_ROW)
                handles.append(pltpu.async_copy(st_h.at[idx_s.at[sl]], rows_s.at[sl], sem))
                handles.append(pltpu.async_copy(at_h.at[idx_a.at[sl]], rows_a.at[sl], sem))
                handles.append(pltpu.async_copy(rt_h.at[idx_r.at[sl]], rows_r.at[sl], sem))
            for h in handles:
                h.wait()
            pltpu.sync_copy(rows_s, os_h.at[pl.ds(tbase, _CTOK)])
            pltpu.sync_copy(rows_a, oa_h.at[pl.ds(tbase, _CTOK)])
            pltpu.sync_copy(rows_r, or_h.at[pl.ds(tbase, _CTOK)])
            return carry

        lax.fori_loop(0, _NCH, body, 0)

    return k(sid2, aid2, rid2, song_table, album_table, artist_table)


def _tc_fused(e_s, e_a, e_r, feats, W, b, alpha, bias):
    TB = 2048
    scale = math.sqrt(_DM)

    def body(es_ref, ea_ref, er_ref, f_ref, w_ref, b_ref, al_ref, bi_ref, o_ref):
        en = jnp.dot(f_ref[...], w_ref[...], preferred_element_type=jnp.float32)
        x = jnp.concatenate([es_ref[...], ea_ref[...], er_ref[...]], axis=-1)
        y = x * scale + en + b_ref[...]
        mean = jnp.mean(y, axis=-1, keepdims=True)
        d = y - mean
        var = jnp.sum(d * d, axis=-1, keepdims=True) * (1.0 / (_DM - 1))
        o_ref[...] = al_ref[...] * d / (jnp.sqrt(var) + _EPS) + bi_ref[...]

    return pl.pallas_call(
        body,
        grid=(_N // TB,),
        in_specs=[
            pl.BlockSpec((TB, _DS), lambda i: (i, 0)),
            pl.BlockSpec((TB, _DA), lambda i: (i, 0)),
            pl.BlockSpec((TB, _DR), lambda i: (i, 0)),
            pl.BlockSpec((TB, _NF), lambda i: (i, 0)),
            pl.BlockSpec((_NF, _DM), lambda i: (0, 0)),
            pl.BlockSpec((1, _DM), lambda i: (0, 0)),
            pl.BlockSpec((1, _DM), lambda i: (0, 0)),
            pl.BlockSpec((1, _DM), lambda i: (0, 0)),
        ],
        out_specs=pl.BlockSpec((TB, _DM), lambda i: (i, 0)),
        out_shape=jax.ShapeDtypeStruct((_N, _DM), jnp.float32),
    )(e_s, e_a, e_r, feats, W, b, alpha, bias)


def kernel(song_ids, album_ids, artist_ids, num_features,
           song_table, album_table, artist_table, W_num, b_num, alpha, bias):
    sid2 = song_ids.reshape(_N)
    aid2 = album_ids.reshape(_N)
    rid2 = artist_ids.reshape(_N)
    e_s, e_a, e_r = _sc_gather(sid2, aid2, rid2,
                               song_table, album_table, artist_table)
    out = _tc_fused(e_s, e_a, e_r,
                    num_features.reshape(_N, _NF), W_num,
                    b_num.reshape(1, _DM), alpha.reshape(1, _DM),
                    bias.reshape(1, _DM))
    return out.reshape(_B, _L, _DM)


# speedup vs baseline: 1.9593x; 1.3362x over previous
import functools
import math

import jax
import jax.numpy as jnp
from jax import lax
from jax.experimental import pallas as pl
from jax.experimental.pallas import tpu as pltpu
from jax.experimental.pallas import tpu_sc as plsc

_DS, _DA, _DR = 64, 32, 32
_DM = _DS + _DA + _DR
_NF = 26
_B, _L = 4096, 50
_N = _B * _L
_EPS = 1e-6

_NC, _NSUB = 2, 16
_NW = _NC * _NSUB
_IDS_PER_GATHER = 128
_CROWS = 5
_CTOK = _CROWS * _IDS_PER_GATHER
_TPW = _N // _NW
_NCH = _TPW // _CTOK


def _sc_gather(sid, aid, rid, song_table, album_table, artist_table):
    mesh = plsc.VectorSubcoreMesh(core_axis_name="c", subcore_axis_name="s")

    @functools.partial(
        pl.kernel,
        mesh=mesh,
        compiler_params=pltpu.CompilerParams(use_tc_tiling_on_sc=False),
        out_type=jax.ShapeDtypeStruct((_N, _DM), jnp.float32),
        scratch_types=[
            pltpu.VMEM((_CTOK,), jnp.int32),
            pltpu.VMEM((_CTOK,), jnp.int32),
            pltpu.VMEM((_CTOK,), jnp.int32),
            pltpu.VMEM((_CTOK, _DS), jnp.float32),
            pltpu.VMEM((_CTOK, _DA), jnp.float32),
            pltpu.VMEM((_CTOK, _DR), jnp.float32),
            pltpu.SemaphoreType.DMA,
        ],
    )
    def k(sid_h, aid_h, rid_h, st_h, at_h, rt_h, cat_h,
          idx_s, idx_a, idx_r, rows_s, rows_a, rows_r, sem):
        wid = lax.axis_index("s") * _NC + lax.axis_index("c")

        def body(c, carry):
            tbase = (wid * _NCH + c) * _CTOK
            pltpu.sync_copy(sid_h.at[pl.ds(tbase, _CTOK)], idx_s)
            pltpu.sync_copy(aid_h.at[pl.ds(tbase, _CTOK)], idx_a)
            pltpu.sync_copy(rid_h.at[pl.ds(tbase, _CTOK)], idx_r)
            handles = []
            for j in range(_CROWS):
                sl = pl.ds(j * _IDS_PER_GATHER, _IDS_PER_GATHER)
                handles.append(pltpu.async_copy(st_h.at[idx_s.at[sl]], rows_s.at[sl], sem))
                handles.append(pltpu.async_copy(at_h.at[idx_a.at[sl]], rows_a.at[sl], sem))
                handles.append(pltpu.async_copy(rt_h.at[idx_r.at[sl]], rows_r.at[sl], sem))
            for h in handles:
                h.wait()
            rows = pl.ds(tbase, _CTOK)
            pltpu.sync_copy(rows_s, cat_h.at[rows, pl.ds(0, _DS)])
            pltpu.sync_copy(rows_a, cat_h.at[rows, pl.ds(_DS, _DA)])
            pltpu.sync_copy(rows_r, cat_h.at[rows, pl.ds(_DS + _DA, _DR)])
            return carry

        lax.fori_loop(0, _NCH, body, 0)

    return k(sid, aid, rid, song_table, album_table, artist_table)


def _tc_fused(cat, feats3, W, b, alpha, bias):
    BB = 64
    TB = BB * _L
    scale = math.sqrt(_DM)

    def body(cat_ref, f_ref, w_ref, b_ref, al_ref, bi_ref, o_ref):
        f = f_ref[...]
        en = lax.dot_general(f, w_ref[...],
                             (((2,), (0,)), ((), ())),
                             preferred_element_type=jnp.float32)
        x = cat_ref[...].reshape(BB, _L, _DM)
        y = x * scale + en + b_ref[...].reshape(1, 1, _DM)
        mean = jnp.mean(y, axis=-1, keepdims=True)
        d = y - mean
        var = jnp.sum(d * d, axis=-1, keepdims=True) * (1.0 / (_DM - 1))
        o_ref[...] = (al_ref[...].reshape(1, 1, _DM) * d / (jnp.sqrt(var) + _EPS)
                      + bi_ref[...].reshape(1, 1, _DM))

    return pl.pallas_call(
        body,
        grid=(_B // BB,),
        in_specs=[
            pl.BlockSpec((TB, _DM), lambda i: (i, 0)),
            pl.BlockSpec((BB, _L, _NF), lambda i: (i, 0, 0)),
            pl.BlockSpec((_NF, _DM), lambda i: (0, 0)),
            pl.BlockSpec((1, _DM), lambda i: (0, 0)),
            pl.BlockSpec((1, _DM), lambda i: (0, 0)),
            pl.BlockSpec((1, _DM), lambda i: (0, 0)),
        ],
        out_specs=pl.BlockSpec((BB, _L, _DM), lambda i: (i, 0, 0)),
        out_shape=jax.ShapeDtypeStruct((_B, _L, _DM), jnp.float32),
    )(cat, feats3, W, b, alpha, bias)


def kernel(song_ids, album_ids, artist_ids, num_features,
           song_table, album_table, artist_table, W_num, b_num, alpha, bias):
    cat = _sc_gather(song_ids.reshape(_N), album_ids.reshape(_N),
                     artist_ids.reshape(_N),
                     song_table, album_table, artist_table)
    return _tc_fused(cat, num_features, W_num,
                     b_num.reshape(1, _DM), alpha.reshape(1, _DM),
                     bias.reshape(1, _DM))


# speedup vs baseline: 2.4535x; 1.2522x over previous
import functools
import math

import jax
import jax.numpy as jnp
from jax import lax
from jax.experimental import pallas as pl
from jax.experimental.pallas import tpu as pltpu
from jax.experimental.pallas import tpu_sc as plsc

_DS, _DA, _DR = 64, 32, 32
_DM = _DS + _DA + _DR
_NF = 26
_B, _L = 4096, 50
_N = _B * _L
_EPS = 1e-6

_NC, _NSUB = 2, 16
_NW = _NC * _NSUB
_IDS_PER_GATHER = 128
_CROWS = 5
_CTOK = _CROWS * _IDS_PER_GATHER
_TPW = _N // _NW
_NCH = _TPW // _CTOK


def _sc_gather(sid, aid, rid, song_table, album_table, artist_table):
    mesh = plsc.VectorSubcoreMesh(core_axis_name="c", subcore_axis_name="s")

    @functools.partial(
        pl.kernel,
        mesh=mesh,
        compiler_params=pltpu.CompilerParams(use_tc_tiling_on_sc=False),
        out_type=jax.ShapeDtypeStruct((_N, _DM), jnp.float32),
        scratch_types=[
            pltpu.VMEM((_CTOK,), jnp.int32),
            pltpu.VMEM((_CTOK,), jnp.int32),
            pltpu.VMEM((_CTOK,), jnp.int32),
            pltpu.VMEM((_CTOK, _DS), jnp.float32),
            pltpu.VMEM((_CTOK, _DA), jnp.float32),
            pltpu.VMEM((_CTOK, _DR), jnp.float32),
            pltpu.SemaphoreType.DMA,
        ],
    )
    def k(sid_h, aid_h, rid_h, st_h, at_h, rt_h, cat_h,
          idx_s, idx_a, idx_r, rows_s, rows_a, rows_r, sem):
        wid = lax.axis_index("s") * _NC + lax.axis_index("c")

        def body(c, carry):
            tbase = (wid * _NCH + c) * _CTOK
            pltpu.sync_copy(sid_h.at[pl.ds(tbase, _CTOK)], idx_s)
            pltpu.sync_copy(aid_h.at[pl.ds(tbase, _CTOK)], idx_a)
            pltpu.sync_copy(rid_h.at[pl.ds(tbase, _CTOK)], idx_r)
            handles = []
            for j in range(_CROWS):
                sl = pl.ds(j * _IDS_PER_GATHER, _IDS_PER_GATHER)
                handles.append(pltpu.async_copy(st_h.at[idx_s.at[sl]], rows_s.at[sl], sem))
                handles.append(pltpu.async_copy(at_h.at[idx_a.at[sl]], rows_a.at[sl], sem))
                handles.append(pltpu.async_copy(rt_h.at[idx_r.at[sl]], rows_r.at[sl], sem))
            for h in handles:
                h.wait()
            rows = pl.ds(tbase, _CTOK)
            pltpu.sync_copy(rows_s, cat_h.at[rows, pl.ds(0, _DS)])
            pltpu.sync_copy(rows_a, cat_h.at[rows, pl.ds(_DS, _DA)])
            pltpu.sync_copy(rows_r, cat_h.at[rows, pl.ds(_DS + _DA, _DR)])
            return carry

        lax.fori_loop(0, _NCH, body, 0)

    return k(sid, aid, rid, song_table, album_table, artist_table)


def _tc_fused(cat3, featsT, W, b, alpha, bias):
    BB = 128
    scale = math.sqrt(_DM)

    def body(cat_ref, f_ref, w_ref, b_ref, al_ref, bi_ref, o_ref):
        w = w_ref[...]
        bb = b_ref[...]
        al = al_ref[...]
        bi = bi_ref[...]
        for l in range(_L):
            f_l = f_ref[:, l, :]
            en = lax.dot_general(f_l, w,
                                 (((0,), (0,)), ((), ())),
                                 preferred_element_type=jnp.float32)
            y = cat_ref[l, :, :] * scale + en + bb
            mean = jnp.mean(y, axis=-1, keepdims=True)
            d = y - mean
            var = jnp.sum(d * d, axis=-1, keepdims=True) * (1.0 / (_DM - 1))
            o_ref[l, :, :] = al * d / (jnp.sqrt(var) + _EPS) + bi

    return pl.pallas_call(
        body,
        grid=(_B // BB,),
        in_specs=[
            pl.BlockSpec((_L, BB, _DM), lambda i: (0, i, 0)),
            pl.BlockSpec((_NF, _L, BB), lambda i: (0, 0, i)),
            pl.BlockSpec((_NF, _DM), lambda i: (0, 0)),
            pl.BlockSpec((1, _DM), lambda i: (0, 0)),
            pl.BlockSpec((1, _DM), lambda i: (0, 0)),
            pl.BlockSpec((1, _DM), lambda i: (0, 0)),
        ],
        out_specs=pl.BlockSpec((_L, BB, _DM), lambda i: (0, i, 0)),
        out_shape=jax.ShapeDtypeStruct((_L, _B, _DM), jnp.float32),
    )(cat3, featsT, W, b, alpha, bias)


def kernel(song_ids, album_ids, artist_ids, num_features,
           song_table, album_table, artist_table, W_num, b_num, alpha, bias):
    sid = song_ids.T.reshape(_N)
    aid = album_ids.T.reshape(_N)
    rid = artist_ids.T.reshape(_N)
    featsT = num_features.transpose(2, 1, 0)
    cat = _sc_gather(sid, aid, rid, song_table, album_table, artist_table)
    cat3 = cat.reshape(_L, _B, _DM)
    out_t = _tc_fused(cat3, featsT, W_num,
                      b_num.reshape(1, _DM), alpha.reshape(1, _DM),
                      bias.reshape(1, _DM))
    return out_t.transpose(1, 0, 2)


# speedup vs baseline: 2.4766x; 1.0094x over previous
import functools
import math

import jax
import jax.numpy as jnp
from jax import lax
from jax.experimental import pallas as pl
from jax.experimental.pallas import tpu as pltpu
from jax.experimental.pallas import tpu_sc as plsc

_DS, _DA, _DR = 64, 32, 32
_DM = _DS + _DA + _DR
_NF = 26
_B, _L = 4096, 50
_N = _B * _L
_EPS = 1e-6

_NC, _NSUB = 2, 16
_NW = _NC * _NSUB
_IDS_PER_GATHER = 128
_CROWS = 5
_CTOK = _CROWS * _IDS_PER_GATHER
_TPW = _N // _NW
_NCH = _TPW // _CTOK


def _sc_gather(sid, aid, rid, song_table, album_table, artist_table):
    mesh = plsc.VectorSubcoreMesh(core_axis_name="c", subcore_axis_name="s")

    @functools.partial(
        pl.kernel,
        mesh=mesh,
        compiler_params=pltpu.CompilerParams(use_tc_tiling_on_sc=False),
        out_type=jax.ShapeDtypeStruct((_N, _DM), jnp.float32),
        scratch_types=[
            pltpu.VMEM((_CTOK,), jnp.int32),
            pltpu.VMEM((_CTOK,), jnp.int32),
            pltpu.VMEM((_CTOK,), jnp.int32),
            pltpu.VMEM((_CTOK, _DS), jnp.float32),
            pltpu.VMEM((_CTOK, _DA), jnp.float32),
            pltpu.VMEM((_CTOK, _DR), jnp.float32),
            pltpu.SemaphoreType.DMA,
        ],
    )
    def k(sid_h, aid_h, rid_h, st_h, at_h, rt_h, cat_h,
          idx_s, idx_a, idx_r, rows_s, rows_a, rows_r, sem):
        wid = lax.axis_index("s") * _NC + lax.axis_index("c")

        def body(c, carry):
            tbase = (wid * _NCH + c) * _CTOK
            pltpu.sync_copy(sid_h.at[pl.ds(tbase, _CTOK)], idx_s)
            pltpu.sync_copy(aid_h.at[pl.ds(tbase, _CTOK)], idx_a)
            pltpu.sync_copy(rid_h.at[pl.ds(tbase, _CTOK)], idx_r)
            handles = []
            for j in range(_CROWS):
                sl = pl.ds(j * _IDS_PER_GATHER, _IDS_PER_GATHER)
                handles.append(pltpu.async_copy(st_h.at[idx_s.at[sl]], rows_s.at[sl], sem))
                handles.append(pltpu.async_copy(at_h.at[idx_a.at[sl]], rows_a.at[sl], sem))
                handles.append(pltpu.async_copy(rt_h.at[idx_r.at[sl]], rows_r.at[sl], sem))
            for h in handles:
                h.wait()
            rows = pl.ds(tbase, _CTOK)
            pltpu.sync_copy(rows_s, cat_h.at[rows, pl.ds(0, _DS)])
            pltpu.sync_copy(rows_a, cat_h.at[rows, pl.ds(_DS, _DA)])
            pltpu.sync_copy(rows_r, cat_h.at[rows, pl.ds(_DS + _DA, _DR)])
            return carry

        lax.fori_loop(0, _NCH, body, 0)

    return k(sid, aid, rid, song_table, album_table, artist_table)


def _tc_fused(cat3, featsT, W, b, alpha, bias):
    BB = 256
    scale = math.sqrt(_DM)

    def body(cat_ref, f_ref, w_ref, b_ref, al_ref, bi_ref, o_ref):
        w = w_ref[...]
        bb = b_ref[...]
        al = al_ref[...]
        bi = bi_ref[...]
        for l in range(_L):
            f_l = f_ref[:, l, :]
            en = lax.dot_general(f_l, w,
                                 (((0,), (0,)), ((), ())),
                                 preferred_element_type=jnp.float32)
            y = cat_ref[l, :, :] * scale + en + bb
            mean = jnp.mean(y, axis=-1, keepdims=True)
            d = y - mean
            var = jnp.sum(d * d, axis=-1, keepdims=True) * (1.0 / (_DM - 1))
            o_ref[l, :, :] = al * d / (jnp.sqrt(var) + _EPS) + bi

    return pl.pallas_call(
        body,
        grid=(_B // BB,),
        compiler_params=pltpu.CompilerParams(vmem_limit_bytes=50 * 1024 * 1024),
        in_specs=[
            pl.BlockSpec((_L, BB, _DM), lambda i: (0, i, 0)),
            pl.BlockSpec((_NF, _L, BB), lambda i: (0, 0, i)),
            pl.BlockSpec((_NF, _DM), lambda i: (0, 0)),
            pl.BlockSpec((1, _DM), lambda i: (0, 0)),
            pl.BlockSpec((1, _DM), lambda i: (0, 0)),
            pl.BlockSpec((1, _DM), lambda i: (0, 0)),
        ],
        out_specs=pl.BlockSpec((_L, BB, _DM), lambda i: (0, i, 0)),
        out_shape=jax.ShapeDtypeStruct((_L, _B, _DM), jnp.float32),
    )(cat3, featsT, W, b, alpha, bias)


def kernel(song_ids, album_ids, artist_ids, num_features,
           song_table, album_table, artist_table, W_num, b_num, alpha, bias):
    sid = song_ids.T.reshape(_N)
    aid = album_ids.T.reshape(_N)
    rid = artist_ids.T.reshape(_N)
    featsT = num_features.transpose(2, 1, 0)
    cat = _sc_gather(sid, aid, rid, song_table, album_table, artist_table)
    cat3 = cat.reshape(_L, _B, _DM)
    out_t = _tc_fused(cat3, featsT, W_num,
                      b_num.reshape(1, _DM), alpha.reshape(1, _DM),
                      bias.reshape(1, _DM))
    return out_t.transpose(1, 0, 2)
